# Initial kernel scaffold; baseline (speedup 1.0000x reference)
import functools

import jax
import jax.numpy as jnp
from jax import lax
from jax.experimental import pallas as pl
from jax.experimental.pallas import tpu as pltpu
from jax.experimental.pallas import tpu_sc as plsc

_N = 10000
_E = 320000
_D = 128
_R = 8
_NC = 512
_COLS = 144
_NCORE = 2
_NSUB = 16
_NT = _NCORE * _NSUB
_EPT = _E // _NT
_B = 128
_CAP = 10240
_SEGS = _R * _NC
_DUMP = _SEGS
_ACC_ROWS = 4160
_ZR = 20


def _sc_body(xa_hbm, src_hbm, dst_hbm, et_hbm, cni_hbm,
             sums_ref, xc_ref, mslot_ref,
             cni_v, mark_v, src_v, dst_v, et_v, slist, glist,
             s_stage, g_stage, rows_v, rows16, idx16, st16, zbuf, acc, dsem):
    cid = lax.axis_index("c")
    sid = lax.axis_index("s")
    wid = cid * _NSUB + sid

    pltpu.sync_copy(cni_hbm, cni_v)
    pltpu.sync_copy(src_hbm.at[pl.ds(wid * _EPT, _EPT)], src_v)
    pltpu.sync_copy(dst_hbm.at[pl.ds(wid * _EPT, _EPT)], dst_v)
    pltpu.sync_copy(et_hbm.at[pl.ds(wid * _EPT, _EPT)], et_v)

    def _init_mark(i, _):
        mark_v[pl.ds(i * 16, 16)] = jnp.full((16,), -1, jnp.int32)
        return 0
    lax.fori_loop(0, _N // 16, _init_mark, 0)

    lanes = lax.iota(jnp.int32, 16)

    def _build_mark(kk, _):
        c16 = cni_v[pl.ds(kk * 16, 16)]
        v16 = kk * 16 + lanes
        for l in range(16):
            plsc.store_scatter(mark_v, [c16], v16, mask=lanes == l)
        return 0
    lax.fori_loop(0, _NC // 16, _build_mark, 0)

    for rr in range(_ZR):
        for cc in range(_COLS // 16):
            zbuf[rr, pl.ds(cc * 16, 16)] = jnp.zeros((16,), jnp.float32)
    for k in range(13):
        pltpu.sync_copy(zbuf, acc.at[pl.ds(sid * 260 + k * _ZR, _ZR)])
    plsc.subcore_barrier()

    def _scan(i, off):
        s16 = src_v[pl.ds(i * 16, 16)]
        d16 = dst_v[pl.ds(i * 16, 16)]
        t16 = et_v[pl.ds(i * 16, 16)]
        u = plsc.load_gather(mark_v, [d16])
        m = u >= 0
        seg = t16 * _NC + u
        plsc.store_compressed(slist.at[pl.ds(off, 16)], s16, mask=m)
        plsc.store_compressed(glist.at[pl.ds(off, 16)], seg, mask=m)
        return off + jnp.sum(m.astype(jnp.int32))
    off = lax.fori_loop(0, _EPT // 16, _scan, jnp.int32(0))

    nb = (off + _B - 1) // _B
    pend = nb * _B
    pads = jnp.zeros((16,), jnp.int32)
    padg = jnp.full((16,), _DUMP, jnp.int32)

    def _pad(k, _):
        slist[pl.ds(off + k * 16, 16)] = pads
        glist[pl.ds(off + k * 16, 16)] = padg
        return 0
    lax.fori_loop(0, (pend - off + 15) // 16, _pad, 0)

    def _batch(j, _):
        for t in range(_B // 16):
            s_stage[pl.ds(t * 16, 16)] = slist[pl.ds(j * _B + t * 16, 16)]
            g_stage[pl.ds(t * 16, 16)] = glist[pl.ds(j * _B + t * 16, 16)]
        pltpu.async_copy(xa_hbm.at[s_stage], rows_v, dsem).wait()
        pltpu.sync_copy(rows_v, acc.at[g_stage], add=True)
        return 0
    lax.fori_loop(0, nb, _batch, 0)
    plsc.subcore_barrier()

    pltpu.sync_copy(acc.at[pl.ds(sid * 256, 256)],
                    sums_ref.at[cid, pl.ds(sid * 256, 256)])

    c16 = cni_v[pl.ds(wid * 16, 16)]
    idx16[...] = c16
    u16 = plsc.load_gather(mark_v, [c16])
    st16[...] = u16
    pltpu.sync_copy(st16, mslot_ref.at[pl.ds(wid * 16, 16)])
    pltpu.async_copy(xa_hbm.at[idx16], rows16, dsem).wait()
    pltpu.sync_copy(rows16, xc_ref.at[pl.ds(wid * 16, 16)])


@jax.jit
def _sc_edge_pass(xa, src, dst, et, cni):
    mesh = plsc.VectorSubcoreMesh(core_axis_name="c", subcore_axis_name="s")
    f = pl.kernel(
        _sc_body,
        out_type=[
            jax.ShapeDtypeStruct((_NCORE, _SEGS, _COLS), jnp.float32),
            jax.ShapeDtypeStruct((_NC, _COLS), jnp.float32),
            jax.ShapeDtypeStruct((_NC,), jnp.int32),
        ],
        mesh=mesh,
        scratch_types=[
            pltpu.VMEM((_NC,), jnp.int32),
            pltpu.VMEM((_N,), jnp.int32),
            pltpu.VMEM((_EPT,), jnp.int32),
            pltpu.VMEM((_EPT,), jnp.int32),
            pltpu.VMEM((_EPT,), jnp.int32),
            pltpu.VMEM((_CAP,), jnp.int32),
            pltpu.VMEM((_CAP,), jnp.int32),
            pltpu.VMEM((_B,), jnp.int32),
            pltpu.VMEM((_B,), jnp.int32),
            pltpu.VMEM((_B, _COLS), jnp.float32),
            pltpu.VMEM((16, _COLS), jnp.float32),
            pltpu.VMEM((16,), jnp.int32),
            pltpu.VMEM((16,), jnp.int32),
            pltpu.VMEM((_ZR, _COLS), jnp.float32),
            pltpu.VMEM_SHARED((_ACC_ROWS, _COLS), jnp.float32),
            pltpu.SemaphoreType.DMA,
        ],
    )
    return f(xa, src, dst, et, cni)


def _tc_body(sums_ref, xc_ref, mslot_ref, basis_ref, comp_ref, root_ref,
             bias_ref, Wg_ref, bg_ref, Ws_ref, bs_ref, outg_ref, outs_ref):
    s = sums_ref[0] + sums_ref[1]
    cnt = jnp.sum(s[:, _D:_COLS], axis=1, keepdims=True)
    inv = 1.0 / jnp.maximum(cnt, 1.0)
    mean = s[:, :_D] * inv

    agg = jnp.zeros((_NC, _D), jnp.float32)
    for r in range(_R):
        w_r = comp_ref[r, 0] * basis_ref[0]
        for b in range(1, _R):
            w_r = w_r + comp_ref[r, b] * basis_ref[b]
        agg = agg + jnp.dot(mean[r * _NC:(r + 1) * _NC, :], w_r,
                            preferred_element_type=jnp.float32)

    xc = xc_ref[...][:, :_D]
    h = agg + jnp.dot(xc, root_ref[...],
                      preferred_element_type=jnp.float32) + bias_ref[...]
    h = jnp.maximum(h, 0.0)

    col = lax.broadcasted_iota(jnp.int32, (_NC, _NC), 1)
    p = (mslot_ref[...] == col).astype(jnp.float32)
    hc = jnp.dot(p, h, preferred_element_type=jnp.float32)

    def _head(w_ref, b_ref, out_ref):
        z = jnp.dot(hc, w_ref[...],
                    preferred_element_type=jnp.float32) + b_ref[...]
        m = jnp.max(z, axis=1, keepdims=True)
        lse = jnp.log(jnp.sum(jnp.exp(z - m), axis=1, keepdims=True))
        out_ref[...] = z - m - lse

    _head(Wg_ref, bg_ref, outg_ref)
    _head(Ws_ref, bs_ref, outs_ref)


@jax.jit
def _tc_finish(sums, xc, mslot, basis, comp, root, bias, Wg, bg, Ws, bs):
    g = Wg.shape[1]
    sdim = Ws.shape[1]
    vm = pl.BlockSpec(memory_space=pltpu.VMEM)
    sm = pl.BlockSpec(memory_space=pltpu.SMEM)
    return pl.pallas_call(
        _tc_body,
        out_shape=[
            jax.ShapeDtypeStruct((_NC, g), jnp.float32),
            jax.ShapeDtypeStruct((_NC, sdim), jnp.float32),
        ],
        in_specs=[vm, vm, vm, vm, sm, vm, vm, vm, vm, vm, vm],
        out_specs=[vm, vm],
    )(sums, xc, mslot, basis, comp, root, bias, Wg, bg, Ws, bs)


def kernel(x, edge_index, edge_type, current_node_index, basis, comp, root,
           bias_conv, Wg, bg, Ws, bs):
    x = x.astype(jnp.float32)
    src = edge_index[0].astype(jnp.int32)
    dst = edge_index[1].astype(jnp.int32)
    et = edge_type.astype(jnp.int32)
    cni = current_node_index.astype(jnp.int32)

    xa = jnp.concatenate(
        [x, jnp.ones((_N, 1), jnp.float32),
         jnp.zeros((_N, _COLS - _D - 1), jnp.float32)], axis=1)

    sums, xc, mslot = _sc_edge_pass(xa, src, dst, et, cni)

    return tuple(_tc_finish(
        sums, xc, mslot.reshape(_NC, 1), basis, comp, root,
        bias_conv.reshape(1, _D), Wg, bg.reshape(1, -1), Ws,
        bs.reshape(1, -1)))


# speedup vs baseline: 36.6222x; 36.6222x over previous
import functools

import jax
import jax.numpy as jnp
from jax import lax
from jax.experimental import pallas as pl
from jax.experimental.pallas import tpu as pltpu
from jax.experimental.pallas import tpu_sc as plsc

_N = 10000
_E = 320000
_D = 128
_R = 8
_NC = 512
_COLS = 144
_NCORE = 2
_NSUB = 16
_NT = _NCORE * _NSUB
_EPT = _E // _NT
_B = 128
_CAP = 10240
_SEGS = _R * _NC
_DUMP = _SEGS
_ACC_ROWS = 4224
_ZR = 24


def _sc_body(xa_hbm, src_hbm, dst_hbm, et_hbm, cni_hbm,
             sums_ref, xc_ref, mslot_ref,
             cni_v, mark_v, src_v, dst_v, et_v, slist, glist,
             s_stage, g_stage, rows_v, rows16, idx16, st16, zbuf, acc, dsem):
    cid = lax.axis_index("c")
    sid = lax.axis_index("s")
    wid = cid * _NSUB + sid

    pltpu.sync_copy(cni_hbm, cni_v)
    pltpu.sync_copy(src_hbm.at[pl.ds(wid * _EPT, _EPT)], src_v)
    pltpu.sync_copy(dst_hbm.at[pl.ds(wid * _EPT, _EPT)], dst_v)
    pltpu.sync_copy(et_hbm.at[pl.ds(wid * _EPT, _EPT)], et_v)

    def _init_mark(i, _):
        mark_v[pl.ds(i * 16, 16)] = jnp.full((16,), -1, jnp.int32)
        return 0
    lax.fori_loop(0, _N // 16, _init_mark, 0)

    lanes = lax.iota(jnp.int32, 16)

    def _build_mark(kk, _):
        c16 = cni_v[pl.ds(kk * 16, 16)]
        v16 = kk * 16 + lanes
        for l in range(16):
            plsc.store_scatter(mark_v, [c16], v16, mask=lanes == l)
        return 0
    lax.fori_loop(0, _NC // 16, _build_mark, 0)

    for rr in range(_ZR):
        for cc in range(_COLS // 16):
            zbuf[rr, pl.ds(cc * 16, 16)] = jnp.zeros((16,), jnp.float32)
    for k in range(11):
        pltpu.sync_copy(zbuf, acc.at[pl.ds(sid * 264 + k * _ZR, _ZR)])
    plsc.subcore_barrier()

    def _scan(i, off):
        s16 = src_v[pl.ds(i * 16, 16)]
        d16 = dst_v[pl.ds(i * 16, 16)]
        t16 = et_v[pl.ds(i * 16, 16)]
        u = plsc.load_gather(mark_v, [d16])
        m = u >= 0
        seg = t16 * _NC + u
        plsc.store_compressed(slist.at[pl.ds(off, 16)], s16, mask=m)
        plsc.store_compressed(glist.at[pl.ds(off, 16)], seg, mask=m)
        return off + jnp.sum(m.astype(jnp.int32))
    off = lax.fori_loop(0, _EPT // 16, _scan, jnp.int32(0))

    nb = (off + _B - 1) // _B
    pend = nb * _B
    pads = jnp.zeros((16,), jnp.int32)
    padg = jnp.full((16,), _DUMP, jnp.int32)

    def _pad(k, _):
        slist[pl.ds(off + k * 16, 16)] = pads
        glist[pl.ds(off + k * 16, 16)] = padg
        return 0
    lax.fori_loop(0, (pend - off + 15) // 16, _pad, 0)

    def _batch(j, _):
        for t in range(_B // 16):
            s_stage[pl.ds(t * 16, 16)] = slist[pl.ds(j * _B + t * 16, 16)]
            g_stage[pl.ds(t * 16, 16)] = glist[pl.ds(j * _B + t * 16, 16)]
        pltpu.async_copy(xa_hbm.at[s_stage], rows_v, dsem).wait()
        pltpu.sync_copy(rows_v, acc.at[g_stage], add=True)
        return 0
    lax.fori_loop(0, nb, _batch, 0)
    plsc.subcore_barrier()

    pltpu.sync_copy(acc.at[pl.ds(sid * 256, 256)],
                    sums_ref.at[cid, pl.ds(sid * 256, 256)])

    c16 = cni_v[pl.ds(wid * 16, 16)]
    idx16[...] = c16
    u16 = plsc.load_gather(mark_v, [c16])
    st16[...] = u16
    pltpu.sync_copy(st16, mslot_ref.at[pl.ds(wid * 16, 16)])
    pltpu.async_copy(xa_hbm.at[idx16], rows16, dsem).wait()
    pltpu.sync_copy(rows16, xc_ref.at[pl.ds(wid * 16, 16)])


@jax.jit
def _sc_edge_pass(xa, src, dst, et, cni):
    mesh = plsc.VectorSubcoreMesh(core_axis_name="c", subcore_axis_name="s")
    f = pl.kernel(
        _sc_body,
        out_type=[
            jax.ShapeDtypeStruct((_NCORE, _SEGS, _COLS), jnp.float32),
            jax.ShapeDtypeStruct((_NC, _COLS), jnp.float32),
            jax.ShapeDtypeStruct((_NC,), jnp.int32),
        ],
        mesh=mesh,
        compiler_params=pltpu.CompilerParams(
            needs_layout_passes=False, use_tc_tiling_on_sc=False),
        scratch_types=[
            pltpu.VMEM((_NC,), jnp.int32),
            pltpu.VMEM((_N,), jnp.int32),
            pltpu.VMEM((_EPT,), jnp.int32),
            pltpu.VMEM((_EPT,), jnp.int32),
            pltpu.VMEM((_EPT,), jnp.int32),
            pltpu.VMEM((_CAP,), jnp.int32),
            pltpu.VMEM((_CAP,), jnp.int32),
            pltpu.VMEM((_B,), jnp.int32),
            pltpu.VMEM((_B,), jnp.int32),
            pltpu.VMEM((_B, _COLS), jnp.float32),
            pltpu.VMEM((16, _COLS), jnp.float32),
            pltpu.VMEM((16,), jnp.int32),
            pltpu.VMEM((16,), jnp.int32),
            pltpu.VMEM((_ZR, _COLS), jnp.float32),
            pltpu.VMEM_SHARED((_ACC_ROWS, _COLS), jnp.float32),
            pltpu.SemaphoreType.DMA,
        ],
    )
    return f(xa, src, dst, et, cni)


def _tc_body(sums_ref, xc_ref, mslot_ref, basis_ref, comp_ref, root_ref,
             bias_ref, Wg_ref, bg_ref, Ws_ref, bs_ref, outg_ref, outs_ref):
    s = sums_ref[0] + sums_ref[1]
    cnt = jnp.sum(s[:, _D:_COLS], axis=1, keepdims=True)
    inv = 1.0 / jnp.maximum(cnt, 1.0)
    mean = s[:, :_D] * inv

    agg = jnp.zeros((_NC, _D), jnp.float32)
    for r in range(_R):
        w_r = comp_ref[r, 0] * basis_ref[0]
        for b in range(1, _R):
            w_r = w_r + comp_ref[r, b] * basis_ref[b]
        agg = agg + jnp.dot(mean[r * _NC:(r + 1) * _NC, :], w_r,
                            preferred_element_type=jnp.float32)

    xc = xc_ref[...][:, :_D]
    h = agg + jnp.dot(xc, root_ref[...],
                      preferred_element_type=jnp.float32) + bias_ref[...]
    h = jnp.maximum(h, 0.0)

    col = lax.broadcasted_iota(jnp.int32, (_NC, _NC), 1)
    p = (mslot_ref[...] == col).astype(jnp.float32)
    hc = jnp.dot(p, h, preferred_element_type=jnp.float32)

    def _head(w_ref, b_ref, out_ref):
        z = jnp.dot(hc, w_ref[...],
                    preferred_element_type=jnp.float32) + b_ref[...]
        m = jnp.max(z, axis=1, keepdims=True)
        lse = jnp.log(jnp.sum(jnp.exp(z - m), axis=1, keepdims=True))
        out_ref[...] = z - m - lse

    _head(Wg_ref, bg_ref, outg_ref)
    _head(Ws_ref, bs_ref, outs_ref)


@jax.jit
def _tc_finish(sums, xc, mslot, basis, comp, root, bias, Wg, bg, Ws, bs):
    g = Wg.shape[1]
    sdim = Ws.shape[1]
    vm = pl.BlockSpec(memory_space=pltpu.VMEM)
    sm = pl.BlockSpec(memory_space=pltpu.SMEM)
    return pl.pallas_call(
        _tc_body,
        out_shape=[
            jax.ShapeDtypeStruct((_NC, g), jnp.float32),
            jax.ShapeDtypeStruct((_NC, sdim), jnp.float32),
        ],
        in_specs=[vm, vm, vm, vm, sm, vm, vm, vm, vm, vm, vm],
        out_specs=[vm, vm],
    )(sums, xc, mslot, basis, comp, root, bias, Wg, bg, Ws, bs)


def kernel(x, edge_index, edge_type, current_node_index, basis, comp, root,
           bias_conv, Wg, bg, Ws, bs):
    x = x.astype(jnp.float32)
    src = edge_index[0].astype(jnp.int32)
    dst = edge_index[1].astype(jnp.int32)
    et = edge_type.astype(jnp.int32)
    cni = current_node_index.astype(jnp.int32)

    xa = jnp.concatenate(
        [x, jnp.ones((_N, 1), jnp.float32),
         jnp.zeros((_N, _COLS - _D - 1), jnp.float32)], axis=1)

    sums, xc, mslot = _sc_edge_pass(xa, src, dst, et, cni)

    return tuple(_tc_finish(
        sums, xc, mslot.reshape(_NC, 1), basis, comp, root,
        bias_conv.reshape(1, _D), Wg, bg.reshape(1, -1), Ws,
        bs.reshape(1, -1)))


# speedup vs baseline: 44.4449x; 1.2136x over previous
import jax
import jax.numpy as jnp
from jax import lax
from jax.experimental import pallas as pl
from jax.experimental.pallas import tpu as pltpu
from jax.experimental.pallas import tpu_sc as plsc

_N = 10000
_E = 320000
_D = 128
_R = 8
_NC = 512
_NCORE = 2
_NSUB = 16
_NT = _NCORE * _NSUB
_EPT = _E // _NT
_B = 128
_CAP = 10240
_SEGS = _R * _NC
_DUMP = _SEGS
_ACC_ROWS = 4224
_CW = 16
_ZR = 24


def _sc_body(x_hbm, ei_hbm, et_hbm, cni_hbm,
             sums_ref, cnts_ref, xc_ref, mslot_ref,
             cni_v, mark_v, src_v, dst_v, et_v, slist, glist,
             s_stage, g_stage, rows_v, rows16, idx16, st16,
             zbuf, ones_b, acc, cacc, dsem):
    cid = lax.axis_index("c")
    sid = lax.axis_index("s")
    wid = cid * _NSUB + sid

    pltpu.sync_copy(cni_hbm, cni_v)
    pltpu.sync_copy(ei_hbm.at[0, pl.ds(wid * _EPT, _EPT)], src_v)
    pltpu.sync_copy(ei_hbm.at[1, pl.ds(wid * _EPT, _EPT)], dst_v)
    pltpu.sync_copy(et_hbm.at[pl.ds(wid * _EPT, _EPT)], et_v)

    def _init_mark(i, _):
        mark_v[pl.ds(i * 16, 16)] = jnp.full((16,), -1, jnp.int32)
        return 0
    lax.fori_loop(0, _N // 16, _init_mark, 0)

    lanes = lax.iota(jnp.int32, 16)

    def _build_mark(kk, _):
        c16 = cni_v[pl.ds(kk * 16, 16)]
        v16 = kk * 16 + lanes
        for l in range(16):
            plsc.store_scatter(mark_v, [c16], v16, mask=lanes == l)
        return 0
    lax.fori_loop(0, _NC // 16, _build_mark, 0)

    for rr in range(_ZR):
        for cc in range(_D // 16):
            zbuf[rr, pl.ds(cc * 16, 16)] = jnp.zeros((16,), jnp.float32)
    for rr in range(_B):
        ones_b[rr, pl.ds(0, _CW)] = jnp.ones((16,), jnp.float32)
    for k in range(11):
        pltpu.sync_copy(zbuf, acc.at[pl.ds(sid * 264 + k * _ZR, _ZR)])
        pltpu.sync_copy(zbuf.at[pl.ds(0, _ZR), pl.ds(0, _CW)],
                        cacc.at[pl.ds(sid * 264 + k * _ZR, _ZR)])
    plsc.subcore_barrier()

    def _scan(i, off):
        s16 = src_v[pl.ds(i * 16, 16)]
        d16 = dst_v[pl.ds(i * 16, 16)]
        t16 = et_v[pl.ds(i * 16, 16)]
        u = plsc.load_gather(mark_v, [d16])
        m = u >= 0
        seg = t16 * _NC + u
        plsc.store_compressed(slist.at[pl.ds(off, 16)], s16, mask=m)
        plsc.store_compressed(glist.at[pl.ds(off, 16)], seg, mask=m)
        return off + plsc.all_reduce_population_count(m)[0]
    off = lax.fori_loop(0, _EPT // 16, _scan, jnp.int32(0))

    nb = (off + _B - 1) // _B
    pend = nb * _B
    pads = jnp.zeros((16,), jnp.int32)
    padg = jnp.full((16,), _DUMP, jnp.int32)

    def _pad(k, _):
        slist[pl.ds(off + k * 16, 16)] = pads
        glist[pl.ds(off + k * 16, 16)] = padg
        return 0
    lax.fori_loop(0, (pend - off + 15) // 16, _pad, 0)

    def _batch(j, _):
        for t in range(_B // 16):
            s_stage[pl.ds(t * 16, 16)] = slist[pl.ds(j * _B + t * 16, 16)]
            g_stage[pl.ds(t * 16, 16)] = glist[pl.ds(j * _B + t * 16, 16)]
        pltpu.async_copy(x_hbm.at[s_stage], rows_v, dsem).wait()
        pltpu.sync_copy(rows_v, acc.at[g_stage], add=True)
        pltpu.sync_copy(ones_b, cacc.at[g_stage], add=True)
        return 0
    lax.fori_loop(0, nb, _batch, 0)
    plsc.subcore_barrier()

    pltpu.sync_copy(acc.at[pl.ds(sid * 256, 256)],
                    sums_ref.at[cid, pl.ds(sid * 256, 256)])
    pltpu.sync_copy(cacc.at[pl.ds(sid * 256, 256)],
                    cnts_ref.at[cid, pl.ds(sid * 256, 256)])

    c16 = cni_v[pl.ds(wid * 16, 16)]
    idx16[...] = c16
    u16 = plsc.load_gather(mark_v, [c16])
    st16[...] = u16
    pltpu.sync_copy(st16, mslot_ref.at[pl.ds(wid * 16, 16)])
    pltpu.async_copy(x_hbm.at[idx16], rows16, dsem).wait()
    pltpu.sync_copy(rows16, xc_ref.at[pl.ds(wid * 16, 16)])


@jax.jit
def _sc_edge_pass(x, ei, et, cni):
    mesh = plsc.VectorSubcoreMesh(core_axis_name="c", subcore_axis_name="s")
    f = pl.kernel(
        _sc_body,
        out_type=[
            jax.ShapeDtypeStruct((_NCORE, _SEGS, _D), jnp.float32),
            jax.ShapeDtypeStruct((_NCORE, _SEGS, _CW), jnp.float32),
            jax.ShapeDtypeStruct((_NC, _D), jnp.float32),
            jax.ShapeDtypeStruct((_NC,), jnp.int32),
        ],
        mesh=mesh,
        compiler_params=pltpu.CompilerParams(
            needs_layout_passes=False, use_tc_tiling_on_sc=False),
        scratch_types=[
            pltpu.VMEM((_NC,), jnp.int32),
            pltpu.VMEM((_N,), jnp.int32),
            pltpu.VMEM((_EPT,), jnp.int32),
            pltpu.VMEM((_EPT,), jnp.int32),
            pltpu.VMEM((_EPT,), jnp.int32),
            pltpu.VMEM((_CAP,), jnp.int32),
            pltpu.VMEM((_CAP,), jnp.int32),
            pltpu.VMEM((_B,), jnp.int32),
            pltpu.VMEM((_B,), jnp.int32),
            pltpu.VMEM((_B, _D), jnp.float32),
            pltpu.VMEM((16, _D), jnp.float32),
            pltpu.VMEM((16,), jnp.int32),
            pltpu.VMEM((16,), jnp.int32),
            pltpu.VMEM((_ZR, _D), jnp.float32),
            pltpu.VMEM((_B, _CW), jnp.float32),
            pltpu.VMEM_SHARED((_ACC_ROWS, _D), jnp.float32),
            pltpu.VMEM_SHARED((_ACC_ROWS, _CW), jnp.float32),
            pltpu.SemaphoreType.DMA,
        ],
    )
    return f(x, ei, et, cni)


def _tc_body(sums_ref, cnts_ref, xc_ref, mslot_ref, basis_ref, comp_ref,
             root_ref, bias_ref, Wg_ref, bg_ref, Ws_ref, bs_ref,
             outg_ref, outs_ref):
    s = sums_ref[0] + sums_ref[1]
    c2 = cnts_ref[0] + cnts_ref[1]
    cnt = c2[:, 0:1]
    inv = 1.0 / jnp.maximum(cnt, 1.0)
    mean = s * inv

    agg = jnp.zeros((_NC, _D), jnp.float32)
    for r in range(_R):
        w_r = comp_ref[r, 0] * basis_ref[0]
        for b in range(1, _R):
            w_r = w_r + comp_ref[r, b] * basis_ref[b]
        agg = agg + jnp.dot(mean[r * _NC:(r + 1) * _NC, :], w_r,
                            preferred_element_type=jnp.float32)

    h = agg + jnp.dot(xc_ref[...], root_ref[...],
                      preferred_element_type=jnp.float32) + bias_ref[...]
    h = jnp.maximum(h, 0.0)

    col = lax.broadcasted_iota(jnp.int32, (_NC, _NC), 1)
    p = (mslot_ref[...] == col).astype(jnp.float32)
    hc = jnp.dot(p, h, preferred_element_type=jnp.float32)

    def _head(w_ref, b_ref, out_ref):
        z = jnp.dot(hc, w_ref[...],
                    preferred_element_type=jnp.float32) + b_ref[...]
        m = jnp.max(z, axis=1, keepdims=True)
        lse = jnp.log(jnp.sum(jnp.exp(z - m), axis=1, keepdims=True))
        out_ref[...] = z - m - lse

    _head(Wg_ref, bg_ref, outg_ref)
    _head(Ws_ref, bs_ref, outs_ref)


@jax.jit
def _tc_finish(sums, cnts, xc, mslot, basis, comp, root, bias, Wg, bg, Ws, bs):
    g = Wg.shape[1]
    sdim = Ws.shape[1]
    vm = pl.BlockSpec(memory_space=pltpu.VMEM)
    sm = pl.BlockSpec(memory_space=pltpu.SMEM)
    return pl.pallas_call(
        _tc_body,
        out_shape=[
            jax.ShapeDtypeStruct((_NC, g), jnp.float32),
            jax.ShapeDtypeStruct((_NC, sdim), jnp.float32),
        ],
        in_specs=[vm, vm, vm, vm, vm, sm, vm, vm, vm, vm, vm, vm],
        out_specs=[vm, vm],
    )(sums, cnts, xc, mslot, basis, comp, root, bias, Wg, bg, Ws, bs)


def kernel(x, edge_index, edge_type, current_node_index, basis, comp, root,
           bias_conv, Wg, bg, Ws, bs):
    x = x.astype(jnp.float32)
    ei = edge_index.astype(jnp.int32)
    et = edge_type.astype(jnp.int32)
    cni = current_node_index.astype(jnp.int32)

    sums, cnts, xc, mslot = _sc_edge_pass(x, ei, et, cni)

    return tuple(_tc_finish(
        sums, cnts, xc, mslot.reshape(_NC, 1), basis, comp, root,
        bias_conv.reshape(1, _D), Wg, bg.reshape(1, -1), Ws,
        bs.reshape(1, -1)))


# speedup vs baseline: 48.0483x; 1.0811x over previous
import jax
import jax.numpy as jnp
from jax import lax
from jax.experimental import pallas as pl
from jax.experimental.pallas import tpu as pltpu
from jax.experimental.pallas import tpu_sc as plsc

_N = 10000
_E = 320000
_D = 128
_R = 8
_NC = 512
_NCORE = 2
_NSUB = 16
_NT = _NCORE * _NSUB
_EPT = _E // _NT
_B = 128
_CAP = 10240
_SEGS = _R * _NC
_DUMP = _SEGS
_ACC_ROWS = 4104
_CW = 16
_ZR = 32
_PK = 16384


def _sc_body(x_hbm, ei_hbm, et_hbm, cni_hbm,
             sums_ref, cnts_ref, xc_ref, mslot_ref,
             cni_v, mark_v, src_v, dst_v, et_v, plist,
             s_stage, g_stage, rows_v, rows16, idx16, st16,
             zbuf, ones_b, acc, cacc, dsem):
    cid = lax.axis_index("c")
    sid = lax.axis_index("s")
    wid = cid * _NSUB + sid

    with jax.named_scope("p_stage"):
        pltpu.sync_copy(cni_hbm, cni_v)
        pltpu.sync_copy(ei_hbm.at[0, pl.ds(wid * _EPT, _EPT)], src_v)
        pltpu.sync_copy(ei_hbm.at[1, pl.ds(wid * _EPT, _EPT)], dst_v)
        pltpu.sync_copy(et_hbm.at[pl.ds(wid * _EPT, _EPT)], et_v)

    with jax.named_scope("p_markinit"):
        neg16 = jnp.full((16,), -1, jnp.int32)

        @plsc.parallel_loop(0, _N // 16, unroll=8)
        def _init_mark(i):
            mark_v[pl.ds(i * 16, 16)] = neg16

    lanes = lax.iota(jnp.int32, 16)

    with jax.named_scope("p_markbuild"):
        def _build_mark(kk, _):
            c16 = cni_v[pl.ds(kk * 16, 16)]
            v16 = kk * 16 + lanes
            for l in range(16):
                plsc.store_scatter(mark_v, [c16], v16, mask=lanes == l)
            return 0
        lax.fori_loop(0, _NC // 16, _build_mark, 0)

    with jax.named_scope("p_zero"):
        for rr in range(_ZR):
            for cc in range(_D // 16):
                zbuf[rr, pl.ds(cc * 16, 16)] = jnp.zeros((16,), jnp.float32)
        for rr in range(_B):
            ones_b[rr, pl.ds(0, _CW)] = jnp.ones((16,), jnp.float32)
        for k in range(8):
            pltpu.sync_copy(zbuf, acc.at[pl.ds(sid * 256 + k * _ZR, _ZR)])
            pltpu.sync_copy(zbuf.at[pl.ds(0, _ZR), pl.ds(0, _CW)],
                            cacc.at[pl.ds(sid * 256 + k * _ZR, _ZR)])

        @pl.when(sid == 0)
        def _():
            pltpu.sync_copy(zbuf.at[pl.ds(0, 8)], acc.at[pl.ds(_SEGS, 8)])
            pltpu.sync_copy(zbuf.at[pl.ds(0, 8), pl.ds(0, _CW)],
                            cacc.at[pl.ds(_SEGS, 8)])
        plsc.subcore_barrier()

    with jax.named_scope("p_scan"):
        @plsc.parallel_loop(0, _EPT // 16, unroll=4,
                            carry=jnp.zeros((16,), jnp.int32))
        def _scan(i, off_v):
            s16 = src_v[pl.ds(i * 16, 16)]
            d16 = dst_v[pl.ds(i * 16, 16)]
            t16 = et_v[pl.ds(i * 16, 16)]
            u = plsc.load_gather(mark_v, [d16])
            m = u >= 0
            pk = (t16 * _NC + u) * _PK + s16
            mi = m.astype(jnp.int32)
            pos = off_v + plsc.cumsum(mi) - mi
            plsc.store_scatter(plist, [pos], pk, mask=m)
            return off_v + plsc.all_reduce_population_count(m)
        off = _scan[0]

    with jax.named_scope("p_pad"):
        nb = (off + _B - 1) // _B
        pend = nb * _B
        padv = jnp.full((16,), _DUMP * _PK, jnp.int32)

        def _pad(k, _):
            plist[pl.ds(off + k * 16, 16)] = padv
            return 0
        lax.fori_loop(0, (pend - off + 15) // 16, _pad, 0)

    with jax.named_scope("p_batch"):
        def _batch2(k, _):
            j0 = k * 2
            descs = []
            for b in range(2):
                jj = j0 + b

                @pl.when(jj < nb)
                def _(b=b, jj=jj):
                    for t in range(_B // 16):
                        pk = plist[pl.ds(jj * _B + t * 16, 16)]
                        s_stage[b, pl.ds(t * 16, 16)] = pk & (_PK - 1)
                        g_stage[b, pl.ds(t * 16, 16)] = (
                            lax.shift_right_logical(pk, 14))
                    pltpu.async_copy(
                        x_hbm.at[s_stage.at[b]], rows_v.at[b], dsem)
            for b in range(2):
                jj = j0 + b

                @pl.when(jj < nb)
                def _(b=b, jj=jj):
                    pltpu.make_async_copy(
                        x_hbm.at[s_stage.at[b]], rows_v.at[b], dsem).wait()
                    pltpu.sync_copy(rows_v.at[b], acc.at[g_stage.at[b]],
                                    add=True)
                    pltpu.sync_copy(ones_b, cacc.at[g_stage.at[b]], add=True)
            return 0
        lax.fori_loop(0, (nb + 1) // 2, _batch2, 0)
        plsc.subcore_barrier()

    with jax.named_scope("p_out"):
        pltpu.sync_copy(acc.at[pl.ds(sid * 256, 256)],
                        sums_ref.at[cid, pl.ds(sid * 256, 256)])
        pltpu.sync_copy(cacc.at[pl.ds(sid * 256, 256)],
                        cnts_ref.at[cid, pl.ds(sid * 256, 256)])

        c16 = cni_v[pl.ds(wid * 16, 16)]
        idx16[...] = c16
        u16 = plsc.load_gather(mark_v, [c16])
        st16[...] = u16
        pltpu.sync_copy(st16, mslot_ref.at[pl.ds(wid * 16, 16)])
        pltpu.async_copy(x_hbm.at[idx16], rows16, dsem).wait()
        pltpu.sync_copy(rows16, xc_ref.at[pl.ds(wid * 16, 16)])


@jax.jit
def _sc_edge_pass(x, ei, et, cni):
    mesh = plsc.VectorSubcoreMesh(core_axis_name="c", subcore_axis_name="s")
    f = pl.kernel(
        _sc_body,
        out_type=[
            jax.ShapeDtypeStruct((_NCORE, _SEGS, _D), jnp.float32),
            jax.ShapeDtypeStruct((_NCORE, _SEGS, _CW), jnp.float32),
            jax.ShapeDtypeStruct((_NC, _D), jnp.float32),
            jax.ShapeDtypeStruct((_NC,), jnp.int32),
        ],
        mesh=mesh,
        compiler_params=pltpu.CompilerParams(
            needs_layout_passes=False, use_tc_tiling_on_sc=False),
        scratch_types=[
            pltpu.VMEM((_NC,), jnp.int32),
            pltpu.VMEM((_N,), jnp.int32),
            pltpu.VMEM((_EPT,), jnp.int32),
            pltpu.VMEM((_EPT,), jnp.int32),
            pltpu.VMEM((_EPT,), jnp.int32),
            pltpu.VMEM((_CAP,), jnp.int32),
            pltpu.VMEM((2, _B), jnp.int32),
            pltpu.VMEM((2, _B), jnp.int32),
            pltpu.VMEM((2, _B, _D), jnp.float32),
            pltpu.VMEM((16, _D), jnp.float32),
            pltpu.VMEM((16,), jnp.int32),
            pltpu.VMEM((16,), jnp.int32),
            pltpu.VMEM((_ZR, _D), jnp.float32),
            pltpu.VMEM((_B, _CW), jnp.float32),
            pltpu.VMEM_SHARED((_ACC_ROWS, _D), jnp.float32),
            pltpu.VMEM_SHARED((_ACC_ROWS, _CW), jnp.float32),
            pltpu.SemaphoreType.DMA,
        ],
    )
    return f(x, ei, et, cni)


def _tc_body(sums_ref, cnts_ref, xc_ref, mslot_ref, basis_ref, comp_ref,
             root_ref, bias_ref, Wg_ref, bg_ref, Ws_ref, bs_ref,
             outg_ref, outs_ref):
    s = sums_ref[0] + sums_ref[1]
    c2 = cnts_ref[0] + cnts_ref[1]
    cnt = c2[:, 0:1]
    inv = 1.0 / jnp.maximum(cnt, 1.0)
    mean = s * inv

    agg = jnp.zeros((_NC, _D), jnp.float32)
    for r in range(_R):
        w_r = comp_ref[r, 0] * basis_ref[0]
        for b in range(1, _R):
            w_r = w_r + comp_ref[r, b] * basis_ref[b]
        agg = agg + jnp.dot(mean[r * _NC:(r + 1) * _NC, :], w_r,
                            preferred_element_type=jnp.float32)

    h = agg + jnp.dot(xc_ref[...], root_ref[...],
                      preferred_element_type=jnp.float32) + bias_ref[...]
    h = jnp.maximum(h, 0.0)

    col = lax.broadcasted_iota(jnp.int32, (_NC, _NC), 1)
    p = (mslot_ref[...] == col).astype(jnp.float32)
    hc = jnp.dot(p, h, preferred_element_type=jnp.float32)

    def _head(w_ref, b_ref, out_ref):
        z = jnp.dot(hc, w_ref[...],
                    preferred_element_type=jnp.float32) + b_ref[...]
        m = jnp.max(z, axis=1, keepdims=True)
        lse = jnp.log(jnp.sum(jnp.exp(z - m), axis=1, keepdims=True))
        out_ref[...] = z - m - lse

    _head(Wg_ref, bg_ref, outg_ref)
    _head(Ws_ref, bs_ref, outs_ref)


@jax.jit
def _tc_finish(sums, cnts, xc, mslot, basis, comp, root, bias, Wg, bg, Ws, bs):
    g = Wg.shape[1]
    sdim = Ws.shape[1]
    vm = pl.BlockSpec(memory_space=pltpu.VMEM)
    sm = pl.BlockSpec(memory_space=pltpu.SMEM)
    return pl.pallas_call(
        _tc_body,
        out_shape=[
            jax.ShapeDtypeStruct((_NC, g), jnp.float32),
            jax.ShapeDtypeStruct((_NC, sdim), jnp.float32),
        ],
        in_specs=[vm, vm, vm, vm, vm, sm, vm, vm, vm, vm, vm, vm],
        out_specs=[vm, vm],
    )(sums, cnts, xc, mslot, basis, comp, root, bias, Wg, bg, Ws, bs)


def kernel(x, edge_index, edge_type, current_node_index, basis, comp, root,
           bias_conv, Wg, bg, Ws, bs):
    x = x.astype(jnp.float32)
    ei = edge_index.astype(jnp.int32)
    et = edge_type.astype(jnp.int32)
    cni = current_node_index.astype(jnp.int32)

    sums, cnts, xc, mslot = _sc_edge_pass(x, ei, et, cni)

    return tuple(_tc_finish(
        sums, cnts, xc, mslot.reshape(_NC, 1), basis, comp, root,
        bias_conv.reshape(1, _D), Wg, bg.reshape(1, -1), Ws,
        bs.reshape(1, -1)))


# speedup vs baseline: 50.3388x; 1.0477x over previous
import jax
import jax.numpy as jnp
from jax import lax
from jax.experimental import pallas as pl
from jax.experimental.pallas import tpu as pltpu
from jax.experimental.pallas import tpu_sc as plsc

_N = 10000
_E = 320000
_D = 128
_R = 8
_NC = 512
_NCORE = 2
_NSUB = 16
_NT = _NCORE * _NSUB
_EPT = _E // _NT
_B = 128
_CAP = 10240
_SEGS = _R * _NC
_DUMP = _SEGS
_ACC_ROWS = 4104
_CNT_ROWS = 4112
_CW = 16
_ZR = 32
_PK = 16384


def _sc_body(x_hbm, ei_hbm, et_hbm, cni_hbm,
             sums_ref, cnts_ref, xc_ref, mslot_ref,
             cni_v, mark_v, src_v, dst_v, et_v, plist, cnt_v,
             s_stage, g_stage, rows_v, rows16, idx16, st16,
             zbuf, acc, dsem, ssem):
    cid = lax.axis_index("c")
    sid = lax.axis_index("s")
    wid = cid * _NSUB + sid

    with jax.named_scope("p_stage"):
        pltpu.sync_copy(cni_hbm, cni_v)
        pltpu.sync_copy(ei_hbm.at[0, pl.ds(wid * _EPT, _EPT)], src_v)
        pltpu.sync_copy(ei_hbm.at[1, pl.ds(wid * _EPT, _EPT)], dst_v)
        pltpu.sync_copy(et_hbm.at[pl.ds(wid * _EPT, _EPT)], et_v)

    with jax.named_scope("p_markinit"):
        neg16 = jnp.full((16,), -1, jnp.int32)

        @plsc.parallel_loop(0, _N // 16, unroll=8)
        def _init_mark(i):
            mark_v[pl.ds(i * 16, 16)] = neg16

    lanes = lax.iota(jnp.int32, 16)

    with jax.named_scope("p_markbuild"):
        def _build_mark(kk, _):
            c16 = cni_v[pl.ds(kk * 16, 16)]
            v16 = kk * 16 + lanes
            for l in range(16):
                plsc.store_scatter(mark_v, [c16], v16, mask=lanes == l)
            return 0
        lax.fori_loop(0, _NC // 16, _build_mark, 0)

    with jax.named_scope("p_zero"):
        for rr in range(_ZR):
            for cc in range(_D // 16):
                zbuf[rr, pl.ds(cc * 16, 16)] = jnp.zeros((16,), jnp.float32)
        zi16 = jnp.zeros((16,), jnp.int32)

        @plsc.parallel_loop(0, _CNT_ROWS // 16, unroll=8)
        def _zc(i):
            cnt_v[pl.ds(i * 16, 16)] = zi16

        for k in range(8):
            pltpu.sync_copy(zbuf, acc.at[pl.ds(sid * 256 + k * _ZR, _ZR)])

        @pl.when(sid == 0)
        def _():
            pltpu.sync_copy(zbuf.at[pl.ds(0, 8)], acc.at[pl.ds(_SEGS, 8)])
        plsc.subcore_barrier()

    with jax.named_scope("p_scan"):
        @plsc.parallel_loop(0, _EPT // 16, unroll=4,
                            carry=jnp.zeros((16,), jnp.int32))
        def _scan(i, off_v):
            s16 = src_v[pl.ds(i * 16, 16)]
            d16 = dst_v[pl.ds(i * 16, 16)]
            t16 = et_v[pl.ds(i * 16, 16)]
            u = plsc.load_gather(mark_v, [d16])
            m = u >= 0
            pk = (t16 * _NC + u) * _PK + s16
            mi = m.astype(jnp.int32)
            pos = off_v + plsc.cumsum(mi) - mi
            plsc.store_scatter(plist, [pos], pk, mask=m)
            return off_v + plsc.all_reduce_population_count(m)
        off = _scan[0]

    with jax.named_scope("p_pad"):
        nb = (off + _B - 1) // _B
        pend = nb * _B
        padv = jnp.full((16,), _DUMP * _PK, jnp.int32)

        def _pad(k, _):
            plist[pl.ds(off + k * 16, 16)] = padv
            return 0
        lax.fori_loop(0, (pend - off + 15) // 16, _pad, 0)

    with jax.named_scope("p_hist"):
        onesv = jnp.ones((16,), jnp.int32)

        def _hist(i, _):
            pk = plist[pl.ds(i * 16, 16)]
            plsc.addupdate_scatter(
                cnt_v, [lax.shift_right_logical(pk, 14)], onesv)
            return 0
        lax.fori_loop(0, pend // 16, _hist, 0)

    with jax.named_scope("p_batch"):
        npair = (nb + 1) // 2

        def _batch2(k, _):
            j0 = k * 2
            for b in range(2):
                jj = j0 + b

                @pl.when(jnp.logical_and(k > 0, jj - 2 < nb))
                def _(b=b):
                    pltpu.make_async_copy(
                        rows_v.at[b], acc.at[g_stage.at[b]], ssem).wait()

                @pl.when(jj < nb)
                def _(b=b, jj=jj):
                    for t in range(_B // 16):
                        pk = plist[pl.ds(jj * _B + t * 16, 16)]
                        s_stage[b, pl.ds(t * 16, 16)] = pk & (_PK - 1)
                        g_stage[b, pl.ds(t * 16, 16)] = (
                            lax.shift_right_logical(pk, 14))
                    pltpu.async_copy(
                        x_hbm.at[s_stage.at[b]], rows_v.at[b], dsem)
            for b in range(2):
                jj = j0 + b

                @pl.when(jj < nb)
                def _(b=b):
                    pltpu.make_async_copy(
                        x_hbm.at[s_stage.at[b]], rows_v.at[b], dsem).wait()
                    pltpu.async_copy(rows_v.at[b], acc.at[g_stage.at[b]],
                                     ssem, add=True)
            return 0
        lax.fori_loop(0, npair, _batch2, 0)
        for b in range(2):
            @pl.when(jnp.logical_and(npair > 0, (npair - 1) * 2 + b < nb))
            def _(b=b):
                pltpu.make_async_copy(
                    rows_v.at[b], acc.at[g_stage.at[b]], ssem).wait()
        plsc.subcore_barrier()

    with jax.named_scope("p_out"):
        pltpu.sync_copy(acc.at[pl.ds(sid * 256, 256)],
                        sums_ref.at[cid, pl.ds(sid * 256, 256)])
        pltpu.sync_copy(cnt_v.at[pl.ds(0, _SEGS)], cnts_ref.at[cid, sid])

        c16 = cni_v[pl.ds(wid * 16, 16)]
        idx16[...] = c16
        u16 = plsc.load_gather(mark_v, [c16])
        st16[...] = u16
        pltpu.sync_copy(st16, mslot_ref.at[pl.ds(wid * 16, 16)])
        pltpu.async_copy(x_hbm.at[idx16], rows16, dsem).wait()
        pltpu.sync_copy(rows16, xc_ref.at[pl.ds(wid * 16, 16)])


@jax.jit
def _sc_edge_pass(x, ei, et, cni):
    mesh = plsc.VectorSubcoreMesh(core_axis_name="c", subcore_axis_name="s")
    f = pl.kernel(
        _sc_body,
        out_type=[
            jax.ShapeDtypeStruct((_NCORE, _SEGS, _D), jnp.float32),
            jax.ShapeDtypeStruct((_NCORE, _NSUB, _SEGS), jnp.int32),
            jax.ShapeDtypeStruct((_NC, _D), jnp.float32),
            jax.ShapeDtypeStruct((_NC,), jnp.int32),
        ],
        mesh=mesh,
        compiler_params=pltpu.CompilerParams(
            needs_layout_passes=False, use_tc_tiling_on_sc=False),
        scratch_types=[
            pltpu.VMEM((_NC,), jnp.int32),
            pltpu.VMEM((_N,), jnp.int32),
            pltpu.VMEM((_EPT,), jnp.int32),
            pltpu.VMEM((_EPT,), jnp.int32),
            pltpu.VMEM((_EPT,), jnp.int32),
            pltpu.VMEM((_CAP,), jnp.int32),
            pltpu.VMEM((_CNT_ROWS,), jnp.int32),
            pltpu.VMEM((2, _B), jnp.int32),
            pltpu.VMEM((2, _B), jnp.int32),
            pltpu.VMEM((2, _B, _D), jnp.float32),
            pltpu.VMEM((16, _D), jnp.float32),
            pltpu.VMEM((16,), jnp.int32),
            pltpu.VMEM((16,), jnp.int32),
            pltpu.VMEM((_ZR, _D), jnp.float32),
            pltpu.VMEM_SHARED((_ACC_ROWS, _D), jnp.float32),
            pltpu.SemaphoreType.DMA,
            pltpu.SemaphoreType.DMA,
        ],
    )
    return f(x, ei, et, cni)


def _tc_body(sums_ref, cnts_ref, xc_ref, mslot_ref, basis_ref, comp_ref,
             root_ref, bias_ref, Wg_ref, bg_ref, Ws_ref, bs_ref,
             outg_ref, outs_ref):
    s = sums_ref[0] + sums_ref[1]
    c32 = cnts_ref[...].astype(jnp.float32)
    cnt = jnp.sum(c32, axis=(0, 1))[:, None]
    inv = 1.0 / jnp.maximum(cnt, 1.0)
    mean = s * inv

    agg = jnp.zeros((_NC, _D), jnp.float32)
    for r in range(_R):
        w_r = comp_ref[r, 0] * basis_ref[0]
        for b in range(1, _R):
            w_r = w_r + comp_ref[r, b] * basis_ref[b]
        agg = agg + jnp.dot(mean[r * _NC:(r + 1) * _NC, :], w_r,
                            preferred_element_type=jnp.float32)

    h = agg + jnp.dot(xc_ref[...], root_ref[...],
                      preferred_element_type=jnp.float32) + bias_ref[...]
    h = jnp.maximum(h, 0.0)

    col = lax.broadcasted_iota(jnp.int32, (_NC, _NC), 1)
    p = (mslot_ref[...] == col).astype(jnp.float32)
    hc = jnp.dot(p, h, preferred_element_type=jnp.float32)

    def _head(w_ref, b_ref, out_ref):
        z = jnp.dot(hc, w_ref[...],
                    preferred_element_type=jnp.float32) + b_ref[...]
        m = jnp.max(z, axis=1, keepdims=True)
        lse = jnp.log(jnp.sum(jnp.exp(z - m), axis=1, keepdims=True))
        out_ref[...] = z - m - lse

    _head(Wg_ref, bg_ref, outg_ref)
    _head(Ws_ref, bs_ref, outs_ref)


@jax.jit
def _tc_finish(sums, cnts, xc, mslot, basis, comp, root, bias, Wg, bg, Ws, bs):
    g = Wg.shape[1]
    sdim = Ws.shape[1]
    vm = pl.BlockSpec(memory_space=pltpu.VMEM)
    sm = pl.BlockSpec(memory_space=pltpu.SMEM)
    return pl.pallas_call(
        _tc_body,
        out_shape=[
            jax.ShapeDtypeStruct((_NC, g), jnp.float32),
            jax.ShapeDtypeStruct((_NC, sdim), jnp.float32),
        ],
        in_specs=[vm, vm, vm, vm, vm, sm, vm, vm, vm, vm, vm, vm],
        out_specs=[vm, vm],
    )(sums, cnts, xc, mslot, basis, comp, root, bias, Wg, bg, Ws, bs)


def kernel(x, edge_index, edge_type, current_node_index, basis, comp, root,
           bias_conv, Wg, bg, Ws, bs):
    x = x.astype(jnp.float32)
    ei = edge_index.astype(jnp.int32)
    et = edge_type.astype(jnp.int32)
    cni = current_node_index.astype(jnp.int32)

    sums, cnts, xc, mslot = _sc_edge_pass(x, ei, et, cni)

    return tuple(_tc_finish(
        sums, cnts, xc, mslot.reshape(_NC, 1), basis, comp, root,
        bias_conv.reshape(1, _D), Wg, bg.reshape(1, -1), Ws,
        bs.reshape(1, -1)))


# speedup vs baseline: 51.0630x; 1.0144x over previous
import jax
import jax.numpy as jnp
from jax import lax
from jax.experimental import pallas as pl
from jax.experimental.pallas import tpu as pltpu
from jax.experimental.pallas import tpu_sc as plsc

_N = 10000
_E = 320000
_D = 128
_R = 8
_NC = 512
_NCORE = 2
_NSUB = 16
_NT = _NCORE * _NSUB
_EPT = _E // _NT
_EPW = 10112
_B = 128
_CAP = 10240
_SEGS = _R * _NC
_DUMP = _SEGS
_ACC_ROWS = 4104
_CNT_ROWS = 4112
_CW = 16
_ZR = 32
_PK = 16384


def _sc_body(x_hbm, ei_hbm, et_hbm, cni_hbm,
             sums_ref, cnts_ref, xc_ref, mslot_ref,
             cni_v, mark_v, src_v, dst_v, et_v, plist, cnt_v,
             s_stage, g_stage, rows_v, rows16, idx16, msl_v,
             zbuf, acc, dsem, ssem):
    cid = lax.axis_index("c")
    sid = lax.axis_index("s")
    wid = cid * _NSUB + sid

    base = wid * _EPT
    a0 = base // 128 * 128
    d16 = (base - a0) // 16
    with jax.named_scope("p_stage"):
        pltpu.sync_copy(cni_hbm, cni_v)
        pltpu.sync_copy(ei_hbm.at[0, pl.ds(a0, _EPW)], src_v)
        pltpu.sync_copy(ei_hbm.at[1, pl.ds(a0, _EPW)], dst_v)
        pltpu.sync_copy(et_hbm.at[pl.ds(a0, _EPW)], et_v)

    with jax.named_scope("p_markinit"):
        neg16 = jnp.full((16,), -1, jnp.int32)

        @plsc.parallel_loop(0, _N // 16, unroll=8)
        def _init_mark(i):
            mark_v[pl.ds(i * 16, 16)] = neg16

    lanes = lax.iota(jnp.int32, 16)

    with jax.named_scope("p_markbuild"):
        def _build_mark(kk, _):
            c16 = cni_v[pl.ds(kk * 16, 16)]
            v16 = kk * 16 + lanes
            for l in range(16):
                plsc.store_scatter(mark_v, [c16], v16, mask=lanes == l)
            return 0
        lax.fori_loop(0, _NC // 16, _build_mark, 0)

    with jax.named_scope("p_zero"):
        for rr in range(_ZR):
            for cc in range(_D // 16):
                zbuf[rr, pl.ds(cc * 16, 16)] = jnp.zeros((16,), jnp.float32)
        zi16 = jnp.zeros((16,), jnp.int32)

        @plsc.parallel_loop(0, _CNT_ROWS // 16, unroll=8)
        def _zc(i):
            cnt_v[pl.ds(i * 16, 16)] = zi16

        for k in range(8):
            pltpu.sync_copy(zbuf, acc.at[pl.ds(sid * 256 + k * _ZR, _ZR)])

        @pl.when(sid == 0)
        def _():
            pltpu.sync_copy(zbuf.at[pl.ds(0, 8)], acc.at[pl.ds(_SEGS, 8)])
        plsc.subcore_barrier()

    with jax.named_scope("p_scan"):
        @plsc.parallel_loop(0, _EPT // 16, unroll=4,
                            carry=jnp.zeros((16,), jnp.int32))
        def _scan(i, off_v):
            s16 = src_v[pl.ds((d16 + i) * 16, 16)]
            dd16 = dst_v[pl.ds((d16 + i) * 16, 16)]
            t16 = et_v[pl.ds((d16 + i) * 16, 16)]
            u = plsc.load_gather(mark_v, [dd16])
            m = u >= 0
            pk = (t16 * _NC + u) * _PK + s16
            mi = m.astype(jnp.int32)
            pos = off_v + plsc.cumsum(mi) - mi
            plsc.store_scatter(plist, [pos], pk, mask=m)
            return off_v + plsc.all_reduce_population_count(m)
        off = _scan[0]

    with jax.named_scope("p_pad"):
        nb = (off + _B - 1) // _B
        pend = nb * _B
        padv = jnp.full((16,), _DUMP * _PK, jnp.int32)

        def _pad(k, _):
            plist[pl.ds(off + k * 16, 16)] = padv
            return 0
        lax.fori_loop(0, (pend - off + 15) // 16, _pad, 0)

    with jax.named_scope("p_hist"):
        onesv = jnp.ones((16,), jnp.int32)

        def _hist(i, _):
            pk = plist[pl.ds(i * 16, 16)]
            plsc.addupdate_scatter(
                cnt_v, [lax.shift_right_logical(pk, 14)], onesv)
            return 0
        lax.fori_loop(0, pend // 16, _hist, 0)

    with jax.named_scope("p_batch"):
        npair = (nb + 1) // 2

        def _batch2(k, _):
            j0 = k * 2
            for b in range(2):
                jj = j0 + b

                @pl.when(jnp.logical_and(k > 0, jj - 2 < nb))
                def _(b=b):
                    pltpu.make_async_copy(
                        rows_v.at[b], acc.at[g_stage.at[b]], ssem).wait()

                @pl.when(jj < nb)
                def _(b=b, jj=jj):
                    for t in range(_B // 16):
                        pk = plist[pl.ds(jj * _B + t * 16, 16)]
                        s_stage[b, pl.ds(t * 16, 16)] = pk & (_PK - 1)
                        g_stage[b, pl.ds(t * 16, 16)] = (
                            lax.shift_right_logical(pk, 14))
                    pltpu.async_copy(
                        x_hbm.at[s_stage.at[b]], rows_v.at[b], dsem)
            for b in range(2):
                jj = j0 + b

                @pl.when(jj < nb)
                def _(b=b):
                    pltpu.make_async_copy(
                        x_hbm.at[s_stage.at[b]], rows_v.at[b], dsem).wait()
                    pltpu.async_copy(rows_v.at[b], acc.at[g_stage.at[b]],
                                     ssem, add=True)
            return 0
        lax.fori_loop(0, npair, _batch2, 0)
        for b in range(2):
            @pl.when(jnp.logical_and(npair > 0, (npair - 1) * 2 + b < nb))
            def _(b=b):
                pltpu.make_async_copy(
                    rows_v.at[b], acc.at[g_stage.at[b]], ssem).wait()
        plsc.subcore_barrier()

    with jax.named_scope("p_out"):
        pltpu.sync_copy(acc.at[pl.ds(sid * 256, 256)],
                        sums_ref.at[cid, pl.ds(sid * 256, 256)])
        pltpu.sync_copy(cnt_v.at[pl.ds(0, _SEGS)],
                        cnts_ref.at[cid, pl.ds(sid * _SEGS, _SEGS)])

        c16 = cni_v[pl.ds(wid * 16, 16)]
        idx16[...] = c16
        pltpu.async_copy(x_hbm.at[idx16], rows16, dsem).wait()
        pltpu.sync_copy(rows16, xc_ref.at[pl.ds(wid * 16, 16)])

        @pl.when(jnp.logical_and(cid == 0, sid == 0))
        def _():
            def _msl(k, _):
                cc = cni_v[pl.ds(k * 16, 16)]
                msl_v[pl.ds(k * 16, 16)] = plsc.load_gather(mark_v, [cc])
                return 0
            lax.fori_loop(0, _NC // 16, _msl, 0)
            pltpu.sync_copy(msl_v, mslot_ref)


@jax.jit
def _sc_edge_pass(x, ei, et, cni):
    mesh = plsc.VectorSubcoreMesh(core_axis_name="c", subcore_axis_name="s")
    f = pl.kernel(
        _sc_body,
        out_type=[
            jax.ShapeDtypeStruct((_NCORE, _SEGS, _D), jnp.float32),
            jax.ShapeDtypeStruct((_NCORE, _NSUB * _SEGS), jnp.int32),
            jax.ShapeDtypeStruct((_NC, _D), jnp.float32),
            jax.ShapeDtypeStruct((_NC,), jnp.int32),
        ],
        mesh=mesh,
        compiler_params=pltpu.CompilerParams(
            needs_layout_passes=False, use_tc_tiling_on_sc=True),
        scratch_types=[
            pltpu.VMEM((_NC,), jnp.int32),
            pltpu.VMEM((_N,), jnp.int32),
            pltpu.VMEM((_EPW,), jnp.int32),
            pltpu.VMEM((_EPW,), jnp.int32),
            pltpu.VMEM((_EPW,), jnp.int32),
            pltpu.VMEM((_CAP,), jnp.int32),
            pltpu.VMEM((_CNT_ROWS,), jnp.int32),
            pltpu.VMEM((2, _B), jnp.int32),
            pltpu.VMEM((2, _B), jnp.int32),
            pltpu.VMEM((2, _B, _D), jnp.float32),
            pltpu.VMEM((16, _D), jnp.float32),
            pltpu.VMEM((16,), jnp.int32),
            pltpu.VMEM((_NC,), jnp.int32),
            pltpu.VMEM((_ZR, _D), jnp.float32),
            pltpu.VMEM_SHARED((_ACC_ROWS, _D), jnp.float32),
            pltpu.SemaphoreType.DMA,
            pltpu.SemaphoreType.DMA,
        ],
    )
    return f(x, ei, et, cni)


def _tc_body(sums_ref, cnts_ref, xc_ref, mslot_ref, basis_ref, comp_ref,
             root_ref, bias_ref, Wg_ref, bg_ref, Ws_ref, bs_ref,
             outg_ref, outs_ref):
    s = sums_ref[0] + sums_ref[1]
    c32 = cnts_ref[...].astype(jnp.float32)
    cnt = jnp.sum(c32, axis=(0, 1))[:, None]
    inv = 1.0 / jnp.maximum(cnt, 1.0)
    mean = s * inv

    agg = jnp.zeros((_NC, _D), jnp.float32)
    for r in range(_R):
        w_r = comp_ref[r, 0] * basis_ref[0]
        for b in range(1, _R):
            w_r = w_r + comp_ref[r, b] * basis_ref[b]
        agg = agg + jnp.dot(mean[r * _NC:(r + 1) * _NC, :], w_r,
                            preferred_element_type=jnp.float32)

    h = agg + jnp.dot(xc_ref[...], root_ref[...],
                      preferred_element_type=jnp.float32) + bias_ref[...]
    h = jnp.maximum(h, 0.0)

    col = lax.broadcasted_iota(jnp.int32, (_NC, _NC), 1)
    p = (mslot_ref[...] == col).astype(jnp.float32)
    hc = jnp.dot(p, h, preferred_element_type=jnp.float32)

    def _head(w_ref, b_ref, out_ref):
        z = jnp.dot(hc, w_ref[...],
                    preferred_element_type=jnp.float32) + b_ref[...]
        m = jnp.max(z, axis=1, keepdims=True)
        lse = jnp.log(jnp.sum(jnp.exp(z - m), axis=1, keepdims=True))
        out_ref[...] = z - m - lse

    _head(Wg_ref, bg_ref, outg_ref)
    _head(Ws_ref, bs_ref, outs_ref)


@jax.jit
def _tc_finish(sums, cnts, xc, mslot, basis, comp, root, bias, Wg, bg, Ws, bs):
    g = Wg.shape[1]
    sdim = Ws.shape[1]
    vm = pl.BlockSpec(memory_space=pltpu.VMEM)
    sm = pl.BlockSpec(memory_space=pltpu.SMEM)
    return pl.pallas_call(
        _tc_body,
        out_shape=[
            jax.ShapeDtypeStruct((_NC, g), jnp.float32),
            jax.ShapeDtypeStruct((_NC, sdim), jnp.float32),
        ],
        in_specs=[vm, vm, vm, vm, vm, sm, vm, vm, vm, vm, vm, vm],
        out_specs=[vm, vm],
    )(sums, cnts, xc, mslot, basis, comp, root, bias, Wg, bg, Ws, bs)


def kernel(x, edge_index, edge_type, current_node_index, basis, comp, root,
           bias_conv, Wg, bg, Ws, bs):
    x = x.astype(jnp.float32)
    ei = edge_index.astype(jnp.int32)
    et = edge_type.astype(jnp.int32)
    cni = current_node_index.astype(jnp.int32)

    sums, cnts, xc, mslot = _sc_edge_pass(x, ei, et, cni)

    return tuple(_tc_finish(
        sums, cnts.reshape(_NCORE, _NSUB, _SEGS), xc,
        mslot.reshape(_NC, 1), basis, comp, root,
        bias_conv.reshape(1, _D), Wg, bg.reshape(1, -1), Ws,
        bs.reshape(1, -1)))


# speedup vs baseline: 65.4348x; 1.2815x over previous
import jax
import jax.numpy as jnp
from jax import lax
from jax.experimental import pallas as pl
from jax.experimental.pallas import tpu as pltpu
from jax.experimental.pallas import tpu_sc as plsc

_N = 10000
_E = 320000
_D = 128
_R = 8
_NC = 512
_NCORE = 2
_NSUB = 16
_NT = _NCORE * _NSUB
_EPT = _E // _NT
_EPW = 10112
_B = 128
_NBUF = 4
_CAP = 10240
_SEGS = _R * _NC
_DUMP = _SEGS
_ACC_ROWS = 4104
_CNT_ROWS = 4112
_ZR = 32
_PK = 16384


def _sc_body(x_hbm, xbf_hbm, ei_hbm, et_hbm, cni_hbm,
             sums_ref, cnts_ref, xc_ref, mslot_ref,
             cni_v, mark_v, src_v, dst_v, et_v, plist, cnt_v,
             s_stage, g_stage, rows_v, rows16, idx16, msl_v,
             zbuf, acc, dsem, ssem):
    cid = lax.axis_index("c")
    sid = lax.axis_index("s")
    wid = cid * _NSUB + sid

    base = wid * _EPT
    a0 = base // 128 * 128
    d16 = (base - a0) // 16
    with jax.named_scope("p_stage"):
        pltpu.sync_copy(cni_hbm, cni_v)
        e1 = pltpu.async_copy(ei_hbm.at[0, pl.ds(a0, _EPW)], src_v, dsem)
        e2 = pltpu.async_copy(ei_hbm.at[1, pl.ds(a0, _EPW)], dst_v, dsem)
        e3 = pltpu.async_copy(et_hbm.at[pl.ds(a0, _EPW)], et_v, dsem)

    with jax.named_scope("p_markinit"):
        neg16 = jnp.full((16,), -1, jnp.int32)

        @plsc.parallel_loop(0, _N // 16, unroll=8)
        def _init_mark(i):
            mark_v[pl.ds(i * 16, 16)] = neg16

    lanes = lax.iota(jnp.int32, 16)

    with jax.named_scope("p_markbuild"):
        def _build_mark(kk, _):
            c16 = cni_v[pl.ds(kk * 16, 16)]
            v16 = kk * 16 + lanes
            for l in range(16):
                plsc.store_scatter(mark_v, [c16], v16, mask=lanes == l)
            return 0
        lax.fori_loop(0, _NC // 16, _build_mark, 0)

    with jax.named_scope("p_zero"):
        for rr in range(_ZR):
            for cc in range(_D // 32):
                zbuf[rr, pl.ds(cc * 32, 32)] = jnp.zeros((32,), jnp.bfloat16)
        zi16 = jnp.zeros((16,), jnp.int32)

        @plsc.parallel_loop(0, _CNT_ROWS // 16, unroll=8)
        def _zc(i):
            cnt_v[pl.ds(i * 16, 16)] = zi16

        for k in range(8):
            pltpu.sync_copy(zbuf, acc.at[pl.ds(sid * 256 + k * _ZR, _ZR)])

        @pl.when(sid == 0)
        def _():
            pltpu.sync_copy(zbuf.at[pl.ds(0, 8)], acc.at[pl.ds(_SEGS, 8)])
        e1.wait()
        e2.wait()
        e3.wait()
        plsc.subcore_barrier()

    with jax.named_scope("p_scan"):
        @plsc.parallel_loop(0, _EPT // 16, unroll=4,
                            carry=jnp.zeros((16,), jnp.int32))
        def _scan(i, off_v):
            s16 = src_v[pl.ds((d16 + i) * 16, 16)]
            dd16 = dst_v[pl.ds((d16 + i) * 16, 16)]
            t16 = et_v[pl.ds((d16 + i) * 16, 16)]
            u = plsc.load_gather(mark_v, [dd16])
            m = u >= 0
            pk = (t16 * _NC + u) * _PK + s16
            mi = m.astype(jnp.int32)
            pos = off_v + plsc.cumsum(mi) - mi
            plsc.store_scatter(plist, [pos], pk, mask=m)
            return off_v + plsc.all_reduce_population_count(m)
        off = _scan[0]

    with jax.named_scope("p_pad"):
        nb = (off + _B - 1) // _B
        pend = nb * _B
        padv = jnp.full((16,), _DUMP * _PK, jnp.int32)

        def _pad(k, _):
            plist[pl.ds(off + k * 16, 16)] = padv
            return 0
        lax.fori_loop(0, (pend - off + 15) // 16, _pad, 0)

    with jax.named_scope("p_hist"):
        onesv = jnp.ones((16,), jnp.int32)

        def _hist(i, _):
            pk = plist[pl.ds(i * 16, 16)]
            plsc.addupdate_scatter(
                cnt_v, [lax.shift_right_logical(pk, 14)], onesv)
            return 0
        lax.fori_loop(0, pend // 16, _hist, 0)

    with jax.named_scope("p_batch"):
        ngrp = (nb + _NBUF - 1) // _NBUF

        def _batchg(k, _):
            j0 = k * _NBUF
            for b in range(_NBUF):
                jj = j0 + b

                @pl.when(jnp.logical_and(k > 0, jj - _NBUF < nb))
                def _(b=b):
                    pltpu.make_async_copy(
                        rows_v.at[b], acc.at[g_stage.at[b]], ssem).wait()

                @pl.when(jj < nb)
                def _(b=b, jj=jj):
                    for t in range(_B // 16):
                        pk = plist[pl.ds(jj * _B + t * 16, 16)]
                        s_stage[b, pl.ds(t * 16, 16)] = pk & (_PK - 1)
                        g_stage[b, pl.ds(t * 16, 16)] = (
                            lax.shift_right_logical(pk, 14))
                    pltpu.async_copy(
                        xbf_hbm.at[s_stage.at[b]], rows_v.at[b], dsem)
            for b in range(_NBUF):
                jj = j0 + b

                @pl.when(jj < nb)
                def _(b=b):
                    pltpu.make_async_copy(
                        xbf_hbm.at[s_stage.at[b]], rows_v.at[b], dsem).wait()
                    pltpu.async_copy(rows_v.at[b], acc.at[g_stage.at[b]],
                                     ssem, add=True)
            return 0
        lax.fori_loop(0, ngrp, _batchg, 0)
        for b in range(_NBUF):
            @pl.when(jnp.logical_and(ngrp > 0, (ngrp - 1) * _NBUF + b < nb))
            def _(b=b):
                pltpu.make_async_copy(
                    rows_v.at[b], acc.at[g_stage.at[b]], ssem).wait()
        plsc.subcore_barrier()

    with jax.named_scope("p_out"):
        pltpu.sync_copy(acc.at[pl.ds(sid * 256, 256)],
                        sums_ref.at[cid, pl.ds(sid * 256, 256)])
        pltpu.sync_copy(cnt_v.at[pl.ds(0, _SEGS)],
                        cnts_ref.at[cid, pl.ds(sid * _SEGS, _SEGS)])

        c16 = cni_v[pl.ds(wid * 16, 16)]
        idx16[...] = c16
        pltpu.async_copy(x_hbm.at[idx16], rows16, dsem).wait()
        pltpu.sync_copy(rows16, xc_ref.at[pl.ds(wid * 16, 16)])

        @pl.when(jnp.logical_and(cid == 0, sid == 0))
        def _():
            def _msl(k, _):
                cc = cni_v[pl.ds(k * 16, 16)]
                msl_v[pl.ds(k * 16, 16)] = plsc.load_gather(mark_v, [cc])
                return 0
            lax.fori_loop(0, _NC // 16, _msl, 0)
            pltpu.sync_copy(msl_v, mslot_ref)


@jax.jit
def _sc_edge_pass(x, xbf, ei, et, cni):
    mesh = plsc.VectorSubcoreMesh(core_axis_name="c", subcore_axis_name="s")
    f = pl.kernel(
        _sc_body,
        out_type=[
            jax.ShapeDtypeStruct((_NCORE, _SEGS, _D), jnp.bfloat16),
            jax.ShapeDtypeStruct((_NCORE, _NSUB * _SEGS), jnp.int32),
            jax.ShapeDtypeStruct((_NC, _D), jnp.float32),
            jax.ShapeDtypeStruct((_NC,), jnp.int32),
        ],
        mesh=mesh,
        compiler_params=pltpu.CompilerParams(
            needs_layout_passes=False, use_tc_tiling_on_sc=False),
        scratch_types=[
            pltpu.VMEM((_NC,), jnp.int32),
            pltpu.VMEM((_N,), jnp.int32),
            pltpu.VMEM((_EPW,), jnp.int32),
            pltpu.VMEM((_EPW,), jnp.int32),
            pltpu.VMEM((_EPW,), jnp.int32),
            pltpu.VMEM((_CAP,), jnp.int32),
            pltpu.VMEM((_CNT_ROWS,), jnp.int32),
            pltpu.VMEM((_NBUF, _B), jnp.int32),
            pltpu.VMEM((_NBUF, _B), jnp.int32),
            pltpu.VMEM((_NBUF, _B, _D), jnp.bfloat16),
            pltpu.VMEM((16, _D), jnp.float32),
            pltpu.VMEM((16,), jnp.int32),
            pltpu.VMEM((_NC,), jnp.int32),
            pltpu.VMEM((_ZR, _D), jnp.bfloat16),
            pltpu.VMEM_SHARED((_ACC_ROWS, _D), jnp.bfloat16),
            pltpu.SemaphoreType.DMA,
            pltpu.SemaphoreType.DMA,
        ],
    )
    return f(x, xbf, ei, et, cni)


def _tc_body(sums_ref, cnts_ref, xc_ref, mslot_ref, basis_ref, comp_ref,
             root_ref, bias_ref, Wg_ref, bg_ref, Ws_ref, bs_ref,
             outg_ref, outs_ref):
    s = (sums_ref[0].astype(jnp.float32)
         + sums_ref[1].astype(jnp.float32))
    c32 = cnts_ref[...].astype(jnp.float32)
    cnt = jnp.sum(c32, axis=(0, 1))[:, None]
    inv = 1.0 / jnp.maximum(cnt, 1.0)
    mean = s * inv

    agg = jnp.zeros((_NC, _D), jnp.float32)
    for r in range(_R):
        w_r = comp_ref[r, 0] * basis_ref[0]
        for b in range(1, _R):
            w_r = w_r + comp_ref[r, b] * basis_ref[b]
        agg = agg + jnp.dot(mean[r * _NC:(r + 1) * _NC, :], w_r,
                            preferred_element_type=jnp.float32)

    h = agg + jnp.dot(xc_ref[...], root_ref[...],
                      preferred_element_type=jnp.float32) + bias_ref[...]
    h = jnp.maximum(h, 0.0)

    row = lax.broadcasted_iota(jnp.int32, (_NC, _NC), 0)
    q = (mslot_ref[...] == row).astype(jnp.float32)
    hc = lax.dot_general(q, h, (((0,), (0,)), ((), ())),
                         preferred_element_type=jnp.float32)

    ones_row = jnp.ones((1, _NC), jnp.float32)

    def _head(wt_ref, b_ref, out_ref):
        z = lax.dot_general(wt_ref[...], hc, (((1,), (1,)), ((), ())),
                            preferred_element_type=jnp.float32)
        z = z + lax.dot_general(b_ref[...], ones_row,
                                (((0,), (0,)), ((), ())),
                                preferred_element_type=jnp.float32)
        m = jnp.max(z, axis=0, keepdims=True)
        lse = jnp.log(jnp.sum(jnp.exp(z - m), axis=0, keepdims=True))
        out_ref[...] = z - m - lse

    _head(Wg_ref, bg_ref, outg_ref)
    _head(Ws_ref, bs_ref, outs_ref)


@jax.jit
def _tc_finish(sums, cnts, xc, mslot, basis, comp, root, bias, Wgt, bg, Wst, bs):
    g = Wgt.shape[0]
    sdim = Wst.shape[0]
    vm = pl.BlockSpec(memory_space=pltpu.VMEM)
    sm = pl.BlockSpec(memory_space=pltpu.SMEM)
    return pl.pallas_call(
        _tc_body,
        out_shape=[
            jax.ShapeDtypeStruct((g, _NC), jnp.float32),
            jax.ShapeDtypeStruct((sdim, _NC), jnp.float32),
        ],
        in_specs=[vm, vm, vm, vm, vm, sm, vm, vm, vm, vm, vm, vm],
        out_specs=[vm, vm],
    )(sums, cnts, xc, mslot, basis, comp, root, bias, Wgt, bg, Wst, bs)


def kernel(x, edge_index, edge_type, current_node_index, basis, comp, root,
           bias_conv, Wg, bg, Ws, bs):
    x = x.astype(jnp.float32)
    ei = edge_index.astype(jnp.int32)
    et = edge_type.astype(jnp.int32)
    cni = current_node_index.astype(jnp.int32)

    xbf = x.astype(jnp.bfloat16)
    sums, cnts, xc, mslot = _sc_edge_pass(x, xbf, ei, et, cni)

    outg_t, outs_t = _tc_finish(
        sums, cnts.reshape(_NCORE, _NSUB, _SEGS), xc,
        mslot.reshape(1, _NC), basis, comp, root,
        bias_conv.reshape(1, _D), Wg.T, bg.reshape(1, -1), Ws.T,
        bs.reshape(1, -1))
    return outg_t.T, outs_t.T
